# baseline (speedup 1.0000x reference)
import functools
import jax
import jax.numpy as jnp
from jax import lax
from jax.experimental import pallas as pl
from jax.experimental.pallas import tpu as pltpu
from jax.experimental.pallas import tpu_sc as plsc

_N, _D, _K = 65536, 256, 1024
_BN = 1024
_NB = _N // _BN

_CHUNK_BLOCKS = (28, 20, 12, 4)
_S = len(_CHUNK_BLOCKS)

_NC, _NS = 2, 16
_NW = _NC * _NS
_C = 128
_KPS = _K // _NS


def _tc_body0(x_ref, cb_ref, sim_ref, lab_ref):
    x = x_ref[...]
    cb = cb_ref[...]
    sim = lax.dot_general(x, cb, (((1,), (1,)), ((), ())),
                          preferred_element_type=jnp.float32)
    sim_ref[...] = sim
    lab_ref[0, 0, :] = jnp.argmax(sim, axis=1).astype(jnp.int32)


def _tc_body_alias(x_ref, cb_ref, simprev_ref, sim_ref, lab_ref):
    del simprev_ref
    _tc_body0(x_ref, cb_ref, sim_ref, lab_ref)


def _tc_chunk(block0, nblocks, first, x, cb, sim_prev):
    x_spec = pl.BlockSpec((_BN, _D), lambda j, i0=block0: (i0 + j, 0))
    cb_spec = pl.BlockSpec((_K, _D), lambda j: (0, 0))
    sim_spec = pl.BlockSpec((_BN, _K), lambda j, i0=block0: (i0 + j, 0))
    lab_spec = pl.BlockSpec((1, 1, _BN), lambda j: (j, 0, 0))
    out_shape = [jax.ShapeDtypeStruct((_N, _K), jnp.float32),
                 jax.ShapeDtypeStruct((nblocks, 1, _BN), jnp.int32)]
    if first:
        return pl.pallas_call(
            _tc_body0, grid=(nblocks,),
            in_specs=[x_spec, cb_spec],
            out_specs=[sim_spec, lab_spec],
            out_shape=out_shape,
        )(x, cb)
    return pl.pallas_call(
        _tc_body_alias, grid=(nblocks,),
        in_specs=[x_spec, cb_spec, pl.BlockSpec(memory_space=pl.ANY)],
        out_specs=[sim_spec, lab_spec],
        out_shape=out_shape,
        input_output_aliases={2: 0},
    )(x, cb, sim_prev)


_sc_mesh = plsc.VectorSubcoreMesh(core_axis_name="c", subcore_axis_name="s")


def _make_sc_gather(chunk_base, nrows):
    rpw = nrows // _NW
    nloop = rpw // _C

    @functools.partial(
        pl.kernel,
        mesh=_sc_mesh,
        out_type=(),
        scratch_types=[
            pltpu.VMEM((rpw,), jnp.int32),
            [pltpu.VMEM((_C, _D), jnp.float32) for _ in range(2)],
            pltpu.VMEM_SHARED((_K, _D), jnp.float32),
            [pltpu.SemaphoreType.DMA for _ in range(2)],
            [pltpu.SemaphoreType.DMA for _ in range(2)],
            pltpu.SemaphoreType.DMA,
        ],
    )
    def _sc_gather(cb_hbm, lab_hbm, out_hbm, idx_v, rows, cb_sh, rsem, wsem,
                   ssem):
        cid = lax.axis_index("c")
        sid = lax.axis_index("s")
        wid = sid * _NC + cid
        w_base = wid * rpw

        s0 = sid * _KPS
        sh = pltpu.async_copy(cb_hbm.at[pl.ds(s0, _KPS)],
                              cb_sh.at[pl.ds(s0, _KPS)], ssem)
        pltpu.sync_copy(lab_hbm.at[pl.ds(w_base, rpw)], idx_v)
        sh.wait()
        plsc.subcore_barrier()

        wh = {}
        for r in range(nloop):
            b = r % 2
            if r >= 2:
                wh[r - 2].wait()

            def fire_group(g, carry):
                v = idx_v[pl.ds(r * _C + g * 16, 16)]
                for j in range(16):
                    pltpu.async_copy(cb_sh.at[pl.ds(v[j], 1)],
                                     rows[b].at[pl.ds(g * 16 + j, 1)],
                                     rsem[b])
                return carry

            lax.fori_loop(0, _C // 16, fire_group, 0)
            pltpu.make_async_copy(cb_hbm.at[pl.ds(0, _C)], rows[b],
                                  rsem[b]).wait()
            wh[r] = pltpu.async_copy(
                rows[b], out_hbm.at[pl.ds(chunk_base + w_base + r * _C, _C)],
                wsem[b])
        for r in range(max(0, nloop - 2), nloop):
            wh[r].wait()

    return _sc_gather


def _alloc_body(o_ref):
    pass


def kernel(input, codebook):
    preds_buf = pl.pallas_call(
        _alloc_body,
        out_specs=pl.BlockSpec(memory_space=pl.ANY),
        out_shape=jax.ShapeDtypeStruct((_N, _D), jnp.float32),
    )()
    preds_ref = jax.new_ref(preds_buf)

    sim = None
    lab_chunks = []
    block0 = 0
    for ci, nblocks in enumerate(_CHUNK_BLOCKS):
        nrows = nblocks * _BN
        sim, lab3 = _tc_chunk(block0, nblocks, ci == 0, input, codebook, sim)
        lab_chunk = lab3.reshape(nrows)
        lab_chunks.append(lab_chunk)
        _make_sc_gather(block0 * _BN, nrows)(codebook, lab_chunk, preds_ref)
        block0 += nblocks

    labels = jnp.concatenate(lab_chunks)
    preds = preds_ref[...]
    return (preds, labels.astype(jnp.int64), sim)


# speedup vs baseline: 1.0103x; 1.0103x over previous
import functools
import jax
import jax.numpy as jnp
from jax import lax
from jax.experimental import pallas as pl
from jax.experimental.pallas import tpu as pltpu
from jax.experimental.pallas import tpu_sc as plsc

_N, _D, _K = 65536, 256, 1024
_BN = 1024
_NB = _N // _BN

_CHUNK_BLOCKS = (20, 20, 16, 8)
_S = len(_CHUNK_BLOCKS)

_NC, _NS = 2, 16
_NW = _NC * _NS
_C = 128
_KPS = _K // _NS


def _tc_body0(x_ref, cb_ref, sim_ref, lab_ref):
    x = x_ref[...]
    cb = cb_ref[...]
    sim = lax.dot_general(x, cb, (((1,), (1,)), ((), ())),
                          preferred_element_type=jnp.float32)
    sim_ref[...] = sim
    lab_ref[0, 0, :] = jnp.argmax(sim, axis=1).astype(jnp.int32)


def _tc_body_alias(x_ref, cb_ref, simprev_ref, sim_ref, lab_ref):
    del simprev_ref
    _tc_body0(x_ref, cb_ref, sim_ref, lab_ref)


def _tc_chunk(block0, nblocks, first, x, cb, sim_prev):
    x_spec = pl.BlockSpec((_BN, _D), lambda j, i0=block0: (i0 + j, 0))
    cb_spec = pl.BlockSpec((_K, _D), lambda j: (0, 0))
    sim_spec = pl.BlockSpec((_BN, _K), lambda j, i0=block0: (i0 + j, 0))
    lab_spec = pl.BlockSpec((1, 1, _BN), lambda j: (j, 0, 0))
    out_shape = [jax.ShapeDtypeStruct((_N, _K), jnp.float32),
                 jax.ShapeDtypeStruct((nblocks, 1, _BN), jnp.int32)]
    if first:
        return pl.pallas_call(
            _tc_body0, grid=(nblocks,),
            in_specs=[x_spec, cb_spec],
            out_specs=[sim_spec, lab_spec],
            out_shape=out_shape,
        )(x, cb)
    return pl.pallas_call(
        _tc_body_alias, grid=(nblocks,),
        in_specs=[x_spec, cb_spec, pl.BlockSpec(memory_space=pl.ANY)],
        out_specs=[sim_spec, lab_spec],
        out_shape=out_shape,
        input_output_aliases={2: 0},
    )(x, cb, sim_prev)


_sc_mesh = plsc.VectorSubcoreMesh(core_axis_name="c", subcore_axis_name="s")


def _make_sc_gather(chunk_base, nrows):
    rpw = nrows // _NW
    nloop = rpw // _C

    @functools.partial(
        pl.kernel,
        mesh=_sc_mesh,
        out_type=(),
        scratch_types=[
            pltpu.VMEM((rpw,), jnp.int32),
            [pltpu.VMEM((_C, _D), jnp.float32) for _ in range(2)],
            pltpu.VMEM_SHARED((_K, _D), jnp.float32),
            [pltpu.SemaphoreType.DMA for _ in range(2)],
            [pltpu.SemaphoreType.DMA for _ in range(2)],
            pltpu.SemaphoreType.DMA,
        ],
    )
    def _sc_gather(cb_hbm, lab_hbm, out_hbm, idx_v, rows, cb_sh, rsem, wsem,
                   ssem):
        cid = lax.axis_index("c")
        sid = lax.axis_index("s")
        wid = sid * _NC + cid
        w_base = wid * rpw

        s0 = sid * _KPS
        sh = pltpu.async_copy(cb_hbm.at[pl.ds(s0, _KPS)],
                              cb_sh.at[pl.ds(s0, _KPS)], ssem)
        pltpu.sync_copy(lab_hbm.at[pl.ds(w_base, rpw)], idx_v)
        sh.wait()
        plsc.subcore_barrier()

        wh = {}
        for r in range(nloop):
            b = r % 2
            if r >= 2:
                wh[r - 2].wait()

            def fire_group(g, carry):
                v = idx_v[pl.ds(r * _C + g * 16, 16)]
                for j in range(16):
                    pltpu.async_copy(cb_sh.at[pl.ds(v[j], 1)],
                                     rows[b].at[pl.ds(g * 16 + j, 1)],
                                     rsem[b])
                return carry

            lax.fori_loop(0, _C // 16, fire_group, 0)
            pltpu.make_async_copy(cb_hbm.at[pl.ds(0, _C)], rows[b],
                                  rsem[b]).wait()
            wh[r] = pltpu.async_copy(
                rows[b], out_hbm.at[pl.ds(chunk_base + w_base + r * _C, _C)],
                wsem[b])
        for r in range(max(0, nloop - 2), nloop):
            wh[r].wait()

    return _sc_gather


def _alloc_body(o_ref):
    pass


def kernel(input, codebook):
    preds_buf = pl.pallas_call(
        _alloc_body,
        out_specs=pl.BlockSpec(memory_space=pl.ANY),
        out_shape=jax.ShapeDtypeStruct((_N, _D), jnp.float32),
    )()
    preds_ref = jax.new_ref(preds_buf)

    sim = None
    lab_chunks = []
    block0 = 0
    for ci, nblocks in enumerate(_CHUNK_BLOCKS):
        nrows = nblocks * _BN
        sim, lab3 = _tc_chunk(block0, nblocks, ci == 0, input, codebook, sim)
        lab_chunk = lab3.reshape(nrows)
        lab_chunks.append(lab_chunk)
        _make_sc_gather(block0 * _BN, nrows)(codebook, lab_chunk, preds_ref)
        block0 += nblocks

    labels = jnp.concatenate(lab_chunks)
    preds = preds_ref[...]
    return (preds, labels.astype(jnp.int64), sim)


# speedup vs baseline: 1.0139x; 1.0035x over previous
import functools
import jax
import jax.numpy as jnp
from jax import lax
from jax.experimental import pallas as pl
from jax.experimental.pallas import tpu as pltpu
from jax.experimental.pallas import tpu_sc as plsc

_N, _D, _K = 65536, 256, 1024
_BN = 1024

_CHUNK_BLOCKS = (20, 20, 16, 8)

_NC, _NS = 2, 16
_NW = _NC * _NS
_C = 128
_KPS = _K // _NS


def _tc_body0(x_ref, cb_ref, sim_ref, lab_ref):
    x = x_ref[...]
    cb = cb_ref[...]
    sim = lax.dot_general(x, cb, (((1,), (1,)), ((), ())),
                          preferred_element_type=jnp.float32)
    sim_ref[...] = sim
    lab_ref[0, 0, :] = jnp.argmax(sim, axis=1).astype(jnp.int32)


def _tc_body_alias(x_ref, cb_ref, simprev_ref, sim_ref, lab_ref):
    del simprev_ref
    _tc_body0(x_ref, cb_ref, sim_ref, lab_ref)


def _tc_chunk(block0, nblocks, first, x, cb, sim_prev):
    x_spec = pl.BlockSpec((_BN, _D), lambda j, i0=block0: (i0 + j, 0))
    cb_spec = pl.BlockSpec((_K, _D), lambda j: (0, 0))
    sim_spec = pl.BlockSpec((_BN, _K), lambda j, i0=block0: (i0 + j, 0))
    lab_spec = pl.BlockSpec((1, 1, _BN), lambda j: (j, 0, 0))
    out_shape = [jax.ShapeDtypeStruct((_N, _K), jnp.float32),
                 jax.ShapeDtypeStruct((nblocks, 1, _BN), jnp.int32)]
    if first:
        return pl.pallas_call(
            _tc_body0, grid=(nblocks,),
            in_specs=[x_spec, cb_spec],
            out_specs=[sim_spec, lab_spec],
            out_shape=out_shape,
        )(x, cb)
    return pl.pallas_call(
        _tc_body_alias, grid=(nblocks,),
        in_specs=[x_spec, cb_spec, pl.BlockSpec(memory_space=pl.ANY)],
        out_specs=[sim_spec, lab_spec],
        out_shape=out_shape,
        input_output_aliases={2: 0},
    )(x, cb, sim_prev)


_sc_mesh = plsc.VectorSubcoreMesh(core_axis_name="c", subcore_axis_name="s")


def _make_sc_gather(chunk_base, nrows):
    rpw = nrows // _NW
    nloop = rpw // _C

    @functools.partial(
        pl.kernel,
        mesh=_sc_mesh,
        out_type=(),
        scratch_types=[
            pltpu.VMEM((rpw,), jnp.int32),
            [pltpu.VMEM((_C, _D), jnp.float32) for _ in range(2)],
            pltpu.VMEM_SHARED((_K, _D), jnp.float32),
            [pltpu.SemaphoreType.DMA for _ in range(2)],
            [pltpu.SemaphoreType.DMA for _ in range(2)],
            pltpu.SemaphoreType.DMA,
        ],
    )
    def _sc_gather(cb_hbm, lab_hbm, out_hbm, idx_v, rows, cb_sh, rsem, wsem,
                   ssem):
        cid = lax.axis_index("c")
        sid = lax.axis_index("s")
        wid = sid * _NC + cid
        w_base = wid * rpw

        s0 = sid * _KPS
        sh = pltpu.async_copy(cb_hbm.at[pl.ds(s0, _KPS)],
                              cb_sh.at[pl.ds(s0, _KPS)], ssem)
        pltpu.sync_copy(lab_hbm.at[pl.ds(w_base, rpw)], idx_v)
        sh.wait()
        plsc.subcore_barrier()

        wh = {}
        for r in range(nloop):
            b = r % 2
            if r >= 2:
                wh[r - 2].wait()

            def fire_group(g, carry):
                v = idx_v[pl.ds(r * _C + g * 16, 16)]
                for j in range(16):
                    pltpu.async_copy(cb_sh.at[pl.ds(v[j], 1)],
                                     rows[b].at[pl.ds(g * 16 + j, 1)],
                                     rsem[b])
                return carry

            lax.fori_loop(0, _C // 16, fire_group, 0)
            pltpu.make_async_copy(cb_hbm.at[pl.ds(0, _C)], rows[b],
                                  rsem[b]).wait()
            wh[r] = pltpu.async_copy(
                rows[b], out_hbm.at[pl.ds(chunk_base + w_base + r * _C, _C)],
                wsem[b])
        for r in range(max(0, nloop - 2), nloop):
            wh[r].wait()

    return _sc_gather


def _alloc_body(o_ref):
    pass


def kernel(input, codebook):
    preds_buf = pl.pallas_call(
        _alloc_body,
        out_specs=pl.BlockSpec(memory_space=pl.ANY),
        out_shape=jax.ShapeDtypeStruct((_N, _D), jnp.float32),
    )()
    preds_ref = jax.new_ref(preds_buf)

    sim = None
    lab_chunks = []
    block0 = 0
    for ci, nblocks in enumerate(_CHUNK_BLOCKS):
        nrows = nblocks * _BN
        sim, lab3 = _tc_chunk(block0, nblocks, ci == 0, input, codebook, sim)
        lab_chunk = lab3.reshape(nrows)
        lab_chunks.append(lab_chunk)
        _make_sc_gather(block0 * _BN, nrows)(codebook, lab_chunk, preds_ref)
        block0 += nblocks

    labels = jnp.concatenate(lab_chunks)
    preds = preds_ref[...]
    return (preds, labels.astype(jnp.int64), sim)
